# baseline (speedup 1.0000x reference)
import functools

import jax
import jax.numpy as jnp
from jax import lax
from jax.experimental import pallas as pl
from jax.experimental.pallas import tpu as pltpu
from jax.experimental.pallas import tpu_sc as plsc

B = 16384
D = 64
SEQ = 20
NNEG = 20
NL = 16

NC, NS = 2, 16
NW = NC * NS
BPW = B // NW
C = 32
NCHUNK = BPW // C
ROWS = C * SEQ


def _sc_body(idx_in_hbm, tgt_hbm, idx_neg_hbm, emb_in_hbm, emb_out_hbm,
             pos_hbm, negp_hbm,
             idx_in_v, idx_neg_v, idx_tgt_v, rows_v, nrows_v, out_rows_v,
             pos_v, negd_v, sem, nsem):
    wid = lax.axis_index("s") * NC + lax.axis_index("c")

    def chunk_body(c, _):
        b0 = wid * BPW + c * C

        pltpu.sync_copy(idx_in_hbm.at[:, pl.ds(b0, C)], idx_in_v)
        pltpu.sync_copy(idx_neg_hbm.at[:, pl.ds(b0, C)], idx_neg_v)
        pltpu.sync_copy(tgt_hbm.at[pl.ds(b0, C)], idx_tgt_v)

        cps = [pltpu.async_copy(emb_in_hbm.at[idx_in_v.at[l]],
                                rows_v.at[pl.ds(l * C, C)], sem)
               for l in range(SEQ)]
        cps.append(pltpu.async_copy(emb_out_hbm.at[idx_tgt_v], out_rows_v, sem))
        ncps = [pltpu.async_copy(emb_out_hbm.at[idx_neg_v.at[n]],
                                 nrows_v.at[pl.ds(n * C, C)], nsem)
                for n in range(NNEG)]
        for cp in cps:
            cp.wait()

        def pos_b(b, _):
            for k in range(D // NL):
                ds = pl.ds(k * NL, NL)
                bag = rows_v[b, ds]
                for l in range(1, SEQ):
                    bag = bag + rows_v[l * C + b, ds]
                prod = jnp.clip(bag * out_rows_v[b, ds], -10.0, 10.0)
                pos_v[pl.ds(b * D + k * NL, NL)] = prod
            return 0

        lax.fori_loop(0, C, pos_b, 0)

        for cp in ncps:
            cp.wait()

        def neg_b(b, _):
            outs = [out_rows_v[b, pl.ds(k * NL, NL)] for k in range(D // NL)]
            for n in range(NNEG):
                acc = nrows_v[n * C + b, pl.ds(0, NL)] * outs[0]
                for k in range(1, D // NL):
                    acc = acc + nrows_v[n * C + b, pl.ds(k * NL, NL)] * outs[k]
                negd_v[pl.ds((b * NNEG + n) * NL, NL)] = acc
            return 0

        lax.fori_loop(0, C, neg_b, 0)

        pltpu.sync_copy(pos_v, pos_hbm.at[pl.ds(b0 * D, C * D)])
        pltpu.sync_copy(negd_v,
                        negp_hbm.at[pl.ds(b0 * NNEG * NL, C * NNEG * NL)])
        return 0

    lax.fori_loop(0, NCHUNK, chunk_body, 0)


@functools.cache
def _sc_kernel():
    return functools.partial(
        pl.kernel,
        out_type=[jax.ShapeDtypeStruct((B * D,), jnp.float32),
                  jax.ShapeDtypeStruct((B * NNEG * NL,), jnp.float32)],
        mesh=plsc.VectorSubcoreMesh(core_axis_name="c", subcore_axis_name="s",
                                    num_cores=NC, num_subcores=NS),
        scratch_types=[
            pltpu.VMEM((SEQ, C), jnp.int32),
            pltpu.VMEM((NNEG, C), jnp.int32),
            pltpu.VMEM((C,), jnp.int32),
            pltpu.VMEM((ROWS, D), jnp.float32),
            pltpu.VMEM((ROWS, D), jnp.float32),
            pltpu.VMEM((C, D), jnp.float32),
            pltpu.VMEM((C * D,), jnp.float32),
            pltpu.VMEM((C * NNEG * NL,), jnp.float32),
            pltpu.SemaphoreType.DMA,
            pltpu.SemaphoreType.DMA,
        ],
        compiler_params=pltpu.CompilerParams(use_tc_tiling_on_sc=False),
    )(_sc_body)


_POS_ROWS = B * D // 128
_NEG_ROWS = B * NNEG * NL // 128
_TC_GRID = 8


def _tc_body(pos_ref, negp_ref, o_ref):
    i = pl.program_id(0)

    @pl.when(i == 0)
    def _():
        o_ref[0, 0] = 0.0

    def logsig_sum(x):
        return jnp.sum(jnp.minimum(x, 0.0) - jnp.log1p(jnp.exp(-jnp.abs(x))))

    r128 = lax.broadcasted_iota(jnp.int32, (128, 8), 0)
    r8 = lax.broadcasted_iota(jnp.int32, (128, 8), 1)
    sel = (r128 // NL == r8).astype(jnp.float32)
    g = lax.dot_general(negp_ref[...], sel, (((1,), (0,)), ((), ())),
                        precision=lax.Precision.HIGHEST)
    negd = jnp.clip(-g, -10.0, 10.0)
    o_ref[0, 0] += logsig_sum(pos_ref[...]) + logsig_sum(negd)

    @pl.when(i == pl.num_programs(0) - 1)
    def _():
        o_ref[0, 0] = -o_ref[0, 0] * (1.0 / B)


_tc_reduce = pl.pallas_call(
    _tc_body,
    grid=(_TC_GRID,),
    in_specs=[pl.BlockSpec((_POS_ROWS // _TC_GRID, 128), lambda i: (i, 0)),
              pl.BlockSpec((_NEG_ROWS // _TC_GRID, 128), lambda i: (i, 0))],
    out_specs=pl.BlockSpec((1, 1), lambda i: (0, 0),
                           memory_space=pltpu.SMEM),
    out_shape=jax.ShapeDtypeStruct((1, 1), jnp.float32),
)


VOCAB = 1000000
_RP_V = 2048
_RP_GRID = -(-VOCAB // _RP_V)
_RP_ROWS = _RP_GRID * _RP_V


def _repack_body(x_ref, o_ref):
    half = _RP_V // 2
    ya = jnp.transpose(x_ref[:, :half])
    yb = jnp.transpose(x_ref[:, half:])
    o_ref[...] = jnp.concatenate([ya, yb], axis=1)


_repack_tc = pl.pallas_call(
    _repack_body,
    grid=(_RP_GRID,),
    in_specs=[pl.BlockSpec((D, _RP_V), lambda i: (0, i))],
    out_specs=pl.BlockSpec((_RP_V // 2, 2 * D), lambda i: (i, 0)),
    out_shape=jax.ShapeDtypeStruct((_RP_ROWS // 2, 2 * D), jnp.float32),
)


def _repack(table):
    z = _repack_tc(table.T)
    return z.reshape(_RP_ROWS * D).reshape(_RP_ROWS, D)


def _remap(v):
    return (v >> 11) * _RP_V + (v & 1023) * 2 + ((v >> 10) & 1)


def kernel(inputs, target_word, negative_samples, emb_in, emb_out):
    idx_in = _remap(inputs.astype(jnp.int32).T)
    idx_neg = _remap(negative_samples.astype(jnp.int32).T)
    tgt = _remap(target_word.astype(jnp.int32))
    emb_in = _repack(emb_in)
    emb_out = _repack(emb_out)
    pos, negp = _sc_kernel()(idx_in, tgt, idx_neg, emb_in, emb_out)
    out = _tc_reduce(pos.reshape(_POS_ROWS, 128), negp.reshape(_NEG_ROWS, 128))
    return out.reshape(())


# speedup vs baseline: 1.5009x; 1.5009x over previous
import functools

import jax
import jax.numpy as jnp
from jax import lax
from jax.experimental import pallas as pl
from jax.experimental.pallas import tpu as pltpu
from jax.experimental.pallas import tpu_sc as plsc

B = 16384
D = 64
SEQ = 20
NNEG = 20
NL = 16

NC, NS = 2, 16
NW = NC * NS
BPW = B // NW
C = 64
NCHUNK = BPW // C
ROWS = C * SEQ


def _sc_body(idx_in_hbm, tgt_hbm, idx_neg_hbm, emb_in_hbm, emb_out_hbm,
             pos_hbm, negp_hbm,
             idx_in_v, idx_neg_v, idx_tgt_v, rows_v, out_rows_v,
             pos_v, negd_v, sem):
    wid = lax.axis_index("s") * NC + lax.axis_index("c")

    def chunk_body(c, _):
        b0 = wid * BPW + c * C

        pltpu.sync_copy(idx_in_hbm.at[:, pl.ds(b0, C)], idx_in_v)
        pltpu.sync_copy(idx_neg_hbm.at[:, pl.ds(b0, C)], idx_neg_v)
        pltpu.sync_copy(tgt_hbm.at[pl.ds(b0, C)], idx_tgt_v)

        cps = [pltpu.async_copy(emb_in_hbm.at[idx_in_v.at[l]],
                                rows_v.at[pl.ds(l * C, C)], sem)
               for l in range(SEQ)]
        cps.append(pltpu.async_copy(emb_out_hbm.at[idx_tgt_v], out_rows_v, sem))
        for cp in cps:
            cp.wait()

        def pos_b(b, _):
            for k in range(D // NL):
                ds = pl.ds(k * NL, NL)
                bag = rows_v[b, ds]
                for l in range(1, SEQ):
                    bag = bag + rows_v[l * C + b, ds]
                prod = jnp.clip(bag * out_rows_v[b, ds], -10.0, 10.0)
                pos_v[pl.ds(b * D + k * NL, NL)] = prod
            return 0

        lax.fori_loop(0, C, pos_b, 0)

        ncps = [pltpu.async_copy(emb_out_hbm.at[idx_neg_v.at[n]],
                                 rows_v.at[pl.ds(n * C, C)], sem)
                for n in range(NNEG)]
        for cp in ncps:
            cp.wait()

        def neg_b(b, _):
            outs = [out_rows_v[b, pl.ds(k * NL, NL)] for k in range(D // NL)]
            for n in range(NNEG):
                acc = rows_v[n * C + b, pl.ds(0, NL)] * outs[0]
                for k in range(1, D // NL):
                    acc = acc + rows_v[n * C + b, pl.ds(k * NL, NL)] * outs[k]
                negd_v[pl.ds((b * NNEG + n) * NL, NL)] = acc
            return 0

        lax.fori_loop(0, C, neg_b, 0)

        pltpu.sync_copy(pos_v, pos_hbm.at[pl.ds(b0 * D, C * D)])
        pltpu.sync_copy(negd_v,
                        negp_hbm.at[pl.ds(b0 * NNEG * NL, C * NNEG * NL)])
        return 0

    lax.fori_loop(0, NCHUNK, chunk_body, 0)


@functools.cache
def _sc_kernel():
    return functools.partial(
        pl.kernel,
        out_type=[jax.ShapeDtypeStruct((B * D,), jnp.float32),
                  jax.ShapeDtypeStruct((B * NNEG * NL,), jnp.float32)],
        mesh=plsc.VectorSubcoreMesh(core_axis_name="c", subcore_axis_name="s",
                                    num_cores=NC, num_subcores=NS),
        scratch_types=[
            pltpu.VMEM((SEQ, C), jnp.int32),
            pltpu.VMEM((NNEG, C), jnp.int32),
            pltpu.VMEM((C,), jnp.int32),
            pltpu.VMEM((ROWS, D), jnp.float32),
            pltpu.VMEM((C, D), jnp.float32),
            pltpu.VMEM((C * D,), jnp.float32),
            pltpu.VMEM((C * NNEG * NL,), jnp.float32),
            pltpu.SemaphoreType.DMA,
        ],
        compiler_params=pltpu.CompilerParams(use_tc_tiling_on_sc=False),
    )(_sc_body)


_POS_ROWS = B * D // 128
_NEG_ROWS = B * NNEG * NL // 128
_TC_GRID = 8


def _tc_body(pos_ref, negp_ref, o_ref):
    i = pl.program_id(0)

    @pl.when(i == 0)
    def _():
        o_ref[0, 0] = 0.0

    def logsig_sum(x):
        return jnp.sum(jnp.minimum(x, 0.0) - jnp.log1p(jnp.exp(-jnp.abs(x))))

    r128 = lax.broadcasted_iota(jnp.int32, (128, 8), 0)
    r8 = lax.broadcasted_iota(jnp.int32, (128, 8), 1)
    sel = (r128 // NL == r8).astype(jnp.float32)
    g = lax.dot_general(negp_ref[...], sel, (((1,), (0,)), ((), ())),
                        precision=lax.Precision.HIGHEST)
    negd = jnp.clip(-g, -10.0, 10.0)
    o_ref[0, 0] += logsig_sum(pos_ref[...]) + logsig_sum(negd)

    @pl.when(i == pl.num_programs(0) - 1)
    def _():
        o_ref[0, 0] = -o_ref[0, 0] * (1.0 / B)


_tc_reduce = pl.pallas_call(
    _tc_body,
    grid=(_TC_GRID,),
    in_specs=[pl.BlockSpec((_POS_ROWS // _TC_GRID, 128), lambda i: (i, 0)),
              pl.BlockSpec((_NEG_ROWS // _TC_GRID, 128), lambda i: (i, 0))],
    out_specs=pl.BlockSpec((1, 1), lambda i: (0, 0),
                           memory_space=pltpu.SMEM),
    out_shape=jax.ShapeDtypeStruct((1, 1), jnp.float32),
)


VOCAB = 1000000
_RP_V = 8192
_RP_GRID = -(-VOCAB // _RP_V)
_RP_ROWS = _RP_GRID * _RP_V


def _repack_body(x_ref, o_ref):
    half = _RP_V // 2
    ya = jnp.transpose(x_ref[:, :half])
    yb = jnp.transpose(x_ref[:, half:])
    o_ref[...] = jnp.concatenate([ya, yb], axis=1)


_repack_tc = pl.pallas_call(
    _repack_body,
    grid=(_RP_GRID,),
    in_specs=[pl.BlockSpec((D, _RP_V), lambda i: (0, i))],
    out_specs=pl.BlockSpec((_RP_V // 2, 2 * D), lambda i: (i, 0)),
    out_shape=jax.ShapeDtypeStruct((_RP_ROWS // 2, 2 * D), jnp.float32),
)


def _repack(table):
    z = _repack_tc(table.T)
    return z.reshape(_RP_ROWS * D).reshape(_RP_ROWS, D)


def _remap(v):
    half = _RP_V // 2
    return (v // _RP_V) * _RP_V + (v % half) * 2 + (v // half) % 2


def kernel(inputs, target_word, negative_samples, emb_in, emb_out):
    idx_in = _remap(inputs.astype(jnp.int32).T)
    idx_neg = _remap(negative_samples.astype(jnp.int32).T)
    tgt = _remap(target_word.astype(jnp.int32))
    emb_in = _repack(emb_in)
    emb_out = _repack(emb_out)
    pos, negp = _sc_kernel()(idx_in, tgt, idx_neg, emb_in, emb_out)
    out = _tc_reduce(pos.reshape(_POS_ROWS, 128), negp.reshape(_NEG_ROWS, 128))
    return out.reshape(())


# speedup vs baseline: 1.7098x; 1.1392x over previous
import functools

import jax
import jax.numpy as jnp
from jax import lax
from jax.experimental import pallas as pl
from jax.experimental.pallas import tpu as pltpu
from jax.experimental.pallas import tpu_sc as plsc

B = 16384
D = 64
SEQ = 20
NNEG = 20
NL = 16

NC, NS = 2, 16
NW = NC * NS
BPW = B // NW
C = 64
NCHUNK = BPW // C
ROWS = C * SEQ


def _sc_body(idx_in_hbm, tgt_hbm, idx_neg_hbm, emb_in_hbm, emb_out_hbm,
             pos_hbm, negp_hbm,
             idx_in_v, idx_neg_v, idx_tgt_v, rows_v, out_rows_v,
             pos_v, negd_v, sem):
    wid = lax.axis_index("s") * NC + lax.axis_index("c")

    def chunk_body(c, _):
        b0 = wid * BPW + c * C

        pltpu.sync_copy(idx_in_hbm.at[:, pl.ds(b0, C)], idx_in_v)
        pltpu.sync_copy(idx_neg_hbm.at[:, pl.ds(b0, C)], idx_neg_v)
        pltpu.sync_copy(tgt_hbm.at[pl.ds(b0, C)], idx_tgt_v)

        cps = [pltpu.async_copy(emb_in_hbm.at[idx_in_v.at[l]],
                                rows_v.at[pl.ds(l * C, C)], sem)
               for l in range(SEQ)]
        cps.append(pltpu.async_copy(emb_out_hbm.at[idx_tgt_v], out_rows_v, sem))
        for cp in cps:
            cp.wait()

        def pos_b(b, _):
            for k in range(D // NL):
                ds = pl.ds(k * NL, NL)
                bag = rows_v[b, ds]
                for l in range(1, SEQ):
                    bag = bag + rows_v[l * C + b, ds]
                prod = jnp.clip(bag * out_rows_v[b, ds], -10.0, 10.0)
                pos_v[pl.ds(b * D + k * NL, NL)] = prod
            return 0

        lax.fori_loop(0, C, pos_b, 0)

        ncps = [pltpu.async_copy(emb_out_hbm.at[idx_neg_v.at[n]],
                                 rows_v.at[pl.ds(n * C, C)], sem)
                for n in range(NNEG)]
        for cp in ncps:
            cp.wait()

        def neg_b(b, _):
            outs = [out_rows_v[b, pl.ds(k * NL, NL)] for k in range(D // NL)]
            for n in range(NNEG):
                acc = rows_v[n * C + b, pl.ds(0, NL)] * outs[0]
                for k in range(1, D // NL):
                    acc = acc + rows_v[n * C + b, pl.ds(k * NL, NL)] * outs[k]
                negd_v[pl.ds((b * NNEG + n) * NL, NL)] = acc
            return 0

        lax.fori_loop(0, C, neg_b, 0)

        pltpu.sync_copy(pos_v, pos_hbm.at[pl.ds(b0 * D, C * D)])
        pltpu.sync_copy(negd_v,
                        negp_hbm.at[pl.ds(b0 * NNEG * NL, C * NNEG * NL)])
        return 0

    lax.fori_loop(0, NCHUNK, chunk_body, 0)


@functools.cache
def _sc_kernel():
    return functools.partial(
        pl.kernel,
        out_type=[jax.ShapeDtypeStruct((B * D,), jnp.float32),
                  jax.ShapeDtypeStruct((B * NNEG * NL,), jnp.float32)],
        mesh=plsc.VectorSubcoreMesh(core_axis_name="c", subcore_axis_name="s",
                                    num_cores=NC, num_subcores=NS),
        scratch_types=[
            pltpu.VMEM((SEQ, C), jnp.int32),
            pltpu.VMEM((NNEG, C), jnp.int32),
            pltpu.VMEM((C,), jnp.int32),
            pltpu.VMEM((ROWS, D), jnp.float32),
            pltpu.VMEM((C, D), jnp.float32),
            pltpu.VMEM((C * D,), jnp.float32),
            pltpu.VMEM((C * NNEG * NL,), jnp.float32),
            pltpu.SemaphoreType.DMA,
        ],
        compiler_params=pltpu.CompilerParams(use_tc_tiling_on_sc=False),
    )(_sc_body)


_POS_ROWS = B * D // 128
_NEG_ROWS = B * NNEG * NL // 128
_TC_GRID = 8


def _tc_body(pos_ref, negp_ref, o_ref):
    i = pl.program_id(0)

    @pl.when(i == 0)
    def _():
        o_ref[0, 0] = 0.0

    def logsig_sum(x):
        return jnp.sum(jnp.minimum(x, 0.0) - jnp.log1p(jnp.exp(-jnp.abs(x))))

    r128 = lax.broadcasted_iota(jnp.int32, (128, 8), 0)
    r8 = lax.broadcasted_iota(jnp.int32, (128, 8), 1)
    sel = (r128 // NL == r8).astype(jnp.float32)
    g = lax.dot_general(negp_ref[...], sel, (((1,), (0,)), ((), ())),
                        precision=lax.Precision.HIGHEST)
    negd = jnp.clip(-g, -10.0, 10.0)
    o_ref[0, 0] += logsig_sum(pos_ref[...]) + logsig_sum(negd)

    @pl.when(i == pl.num_programs(0) - 1)
    def _():
        o_ref[0, 0] = -o_ref[0, 0] * (1.0 / B)


_tc_reduce = pl.pallas_call(
    _tc_body,
    grid=(_TC_GRID,),
    in_specs=[pl.BlockSpec((_POS_ROWS // _TC_GRID, 128), lambda i: (i, 0)),
              pl.BlockSpec((_NEG_ROWS // _TC_GRID, 128), lambda i: (i, 0))],
    out_specs=pl.BlockSpec((1, 1), lambda i: (0, 0),
                           memory_space=pltpu.SMEM),
    out_shape=jax.ShapeDtypeStruct((1, 1), jnp.float32),
)


VOCAB = 1000000
_RP_V = 32768
_RP_GRID = -(-VOCAB // _RP_V)
_RP_ROWS = _RP_GRID * _RP_V


def _repack_body(x_ref, o_ref):
    half = _RP_V // 2
    ya = jnp.transpose(x_ref[:, :half])
    yb = jnp.transpose(x_ref[:, half:])
    o_ref[...] = jnp.concatenate([ya, yb], axis=1)


_repack_tc = pl.pallas_call(
    _repack_body,
    grid=(_RP_GRID,),
    in_specs=[pl.BlockSpec((D, _RP_V), lambda i: (0, i))],
    out_specs=pl.BlockSpec((_RP_V // 2, 2 * D), lambda i: (i, 0)),
    out_shape=jax.ShapeDtypeStruct((_RP_ROWS // 2, 2 * D), jnp.float32),
)


def _repack(table):
    z = _repack_tc(table.T)
    return z.reshape(_RP_ROWS * D).reshape(_RP_ROWS, D)


def _remap(v):
    half = _RP_V // 2
    return (v // _RP_V) * _RP_V + (v % half) * 2 + (v // half) % 2


def kernel(inputs, target_word, negative_samples, emb_in, emb_out):
    idx_in = _remap(inputs.astype(jnp.int32).T)
    idx_neg = _remap(negative_samples.astype(jnp.int32).T)
    tgt = _remap(target_word.astype(jnp.int32))
    emb_in = _repack(emb_in)
    emb_out = _repack(emb_out)
    pos, negp = _sc_kernel()(idx_in, tgt, idx_neg, emb_in, emb_out)
    out = _tc_reduce(pos.reshape(_POS_ROWS, 128), negp.reshape(_NEG_ROWS, 128))
    return out.reshape(())


# speedup vs baseline: 1.8416x; 1.0771x over previous
import functools

import jax
import jax.numpy as jnp
from jax import lax
from jax.experimental import pallas as pl
from jax.experimental.pallas import tpu as pltpu
from jax.experimental.pallas import tpu_sc as plsc

B = 16384
D = 64
SEQ = 20
NNEG = 20
NL = 16

NC, NS = 2, 16
NW = NC * NS
BPW = B // NW
C = 64
NCHUNK = BPW // C
ROWS = C * SEQ


def _sc_bag_body(idx_in_hbm, emb_in_hbm, bags_hbm,
                 idx_in_v, rows_v, bag_v, sem):
    wid = lax.axis_index("s") * NC + lax.axis_index("c")

    def chunk_body(c, _):
        b0 = wid * BPW + c * C
        pltpu.sync_copy(idx_in_hbm.at[:, pl.ds(b0, C)], idx_in_v)
        cps = [pltpu.async_copy(emb_in_hbm.at[idx_in_v.at[l]],
                                rows_v.at[pl.ds(l * C, C)], sem)
               for l in range(SEQ)]
        for cp in cps:
            cp.wait()

        def bag_b(b, _):
            for k in range(D // NL):
                ds = pl.ds(k * NL, NL)
                bag = rows_v[b, ds]
                for l in range(1, SEQ):
                    bag = bag + rows_v[l * C + b, ds]
                bag_v[pl.ds(b * D + k * NL, NL)] = bag
            return 0

        lax.fori_loop(0, C, bag_b, 0)
        pltpu.sync_copy(bag_v, bags_hbm.at[pl.ds(b0 * D, C * D)])
        return 0

    lax.fori_loop(0, NCHUNK, chunk_body, 0)


def _sc_prod_body(tgt_hbm, idx_neg_hbm, emb_out_hbm, bags_hbm,
                  pos_hbm, negp_hbm,
                  idx_neg_v, idx_tgt_v, rows_v, out_rows_v, bag_v,
                  pos_v, negd_v, sem):
    wid = lax.axis_index("s") * NC + lax.axis_index("c")

    def chunk_body(c, _):
        b0 = wid * BPW + c * C
        pltpu.sync_copy(idx_neg_hbm.at[:, pl.ds(b0, C)], idx_neg_v)
        pltpu.sync_copy(tgt_hbm.at[pl.ds(b0, C)], idx_tgt_v)

        cps = [pltpu.async_copy(emb_out_hbm.at[idx_neg_v.at[n]],
                                rows_v.at[pl.ds(n * C, C)], sem)
               for n in range(NNEG)]
        cps.append(pltpu.async_copy(emb_out_hbm.at[idx_tgt_v], out_rows_v, sem))
        pltpu.sync_copy(bags_hbm.at[pl.ds(b0 * D, C * D)], bag_v)
        for cp in cps:
            cp.wait()

        def prod_b(b, _):
            outs = [out_rows_v[b, pl.ds(k * NL, NL)] for k in range(D // NL)]
            for k in range(D // NL):
                prod = jnp.clip(bag_v[pl.ds(b * D + k * NL, NL)] * outs[k],
                                -10.0, 10.0)
                pos_v[pl.ds(b * D + k * NL, NL)] = prod
            for n in range(NNEG):
                acc = rows_v[n * C + b, pl.ds(0, NL)] * outs[0]
                for k in range(1, D // NL):
                    acc = acc + rows_v[n * C + b, pl.ds(k * NL, NL)] * outs[k]
                negd_v[pl.ds((b * NNEG + n) * NL, NL)] = acc
            return 0

        lax.fori_loop(0, C, prod_b, 0)

        pltpu.sync_copy(pos_v, pos_hbm.at[pl.ds(b0 * D, C * D)])
        pltpu.sync_copy(negd_v,
                        negp_hbm.at[pl.ds(b0 * NNEG * NL, C * NNEG * NL)])
        return 0

    lax.fori_loop(0, NCHUNK, chunk_body, 0)


_SC_MESH = dict(core_axis_name="c", subcore_axis_name="s",
                num_cores=NC, num_subcores=NS)


@functools.cache
def _sc_bag_kernel():
    return functools.partial(
        pl.kernel,
        out_type=[jax.ShapeDtypeStruct((B * D,), jnp.float32)],
        mesh=plsc.VectorSubcoreMesh(**_SC_MESH),
        scratch_types=[
            pltpu.VMEM((SEQ, C), jnp.int32),
            pltpu.VMEM((ROWS, D), jnp.float32),
            pltpu.VMEM((C * D,), jnp.float32),
            pltpu.SemaphoreType.DMA,
        ],
        compiler_params=pltpu.CompilerParams(use_tc_tiling_on_sc=False),
    )(_sc_bag_body)


@functools.cache
def _sc_prod_kernel():
    return functools.partial(
        pl.kernel,
        out_type=[jax.ShapeDtypeStruct((B * D,), jnp.float32),
                  jax.ShapeDtypeStruct((B * NNEG * NL,), jnp.float32)],
        mesh=plsc.VectorSubcoreMesh(**_SC_MESH),
        scratch_types=[
            pltpu.VMEM((NNEG, C), jnp.int32),
            pltpu.VMEM((C,), jnp.int32),
            pltpu.VMEM((ROWS, D), jnp.float32),
            pltpu.VMEM((C, D), jnp.float32),
            pltpu.VMEM((C * D,), jnp.float32),
            pltpu.VMEM((C * D,), jnp.float32),
            pltpu.VMEM((C * NNEG * NL,), jnp.float32),
            pltpu.SemaphoreType.DMA,
        ],
        compiler_params=pltpu.CompilerParams(use_tc_tiling_on_sc=False),
    )(_sc_prod_body)


_POS_ROWS = B * D // 128
_NEG_ROWS = B * NNEG * NL // 128
_TC_GRID = 8


def _tc_body(pos_ref, negp_ref, o_ref):
    i = pl.program_id(0)

    @pl.when(i == 0)
    def _():
        o_ref[0, 0] = 0.0

    def logsig_sum(x):
        return jnp.sum(jnp.minimum(x, 0.0) - jnp.log1p(jnp.exp(-jnp.abs(x))))

    r128 = lax.broadcasted_iota(jnp.int32, (128, 8), 0)
    r8 = lax.broadcasted_iota(jnp.int32, (128, 8), 1)
    sel = (r128 // NL == r8).astype(jnp.float32)
    g = lax.dot_general(negp_ref[...], sel, (((1,), (0,)), ((), ())),
                        precision=lax.Precision.HIGHEST)
    negd = jnp.clip(-g, -10.0, 10.0)
    o_ref[0, 0] += logsig_sum(pos_ref[...]) + logsig_sum(negd)

    @pl.when(i == pl.num_programs(0) - 1)
    def _():
        o_ref[0, 0] = -o_ref[0, 0] * (1.0 / B)


_tc_reduce = pl.pallas_call(
    _tc_body,
    grid=(_TC_GRID,),
    in_specs=[pl.BlockSpec((_POS_ROWS // _TC_GRID, 128), lambda i: (i, 0)),
              pl.BlockSpec((_NEG_ROWS // _TC_GRID, 128), lambda i: (i, 0))],
    out_specs=pl.BlockSpec((1, 1), lambda i: (0, 0),
                           memory_space=pltpu.SMEM),
    out_shape=jax.ShapeDtypeStruct((1, 1), jnp.float32),
)


VOCAB = 1000000
_RP_V = 32768
_RP_GRID = -(-VOCAB // _RP_V)
_RP_ROWS = _RP_GRID * _RP_V


def _repack_body(x_ref, o_ref):
    half = _RP_V // 2
    ya = jnp.transpose(x_ref[:, :half])
    yb = jnp.transpose(x_ref[:, half:])
    o_ref[...] = jnp.concatenate([ya, yb], axis=1)


_repack_tc = pl.pallas_call(
    _repack_body,
    grid=(_RP_GRID,),
    in_specs=[pl.BlockSpec((D, _RP_V), lambda i: (0, i))],
    out_specs=pl.BlockSpec((_RP_V // 2, 2 * D), lambda i: (i, 0)),
    out_shape=jax.ShapeDtypeStruct((_RP_ROWS // 2, 2 * D), jnp.float32),
)


def _repack(table):
    z = _repack_tc(table.T)
    return z.reshape(_RP_ROWS * D).reshape(_RP_ROWS, D)


def _remap(v):
    half = _RP_V // 2
    return (v // _RP_V) * _RP_V + (v % half) * 2 + (v // half) % 2


def kernel(inputs, target_word, negative_samples, emb_in, emb_out):
    idx_in = _remap(inputs.astype(jnp.int32).T)
    idx_neg = _remap(negative_samples.astype(jnp.int32).T)
    tgt = _remap(target_word.astype(jnp.int32))
    emb_in = _repack(emb_in)
    (bags,) = _sc_bag_kernel()(idx_in, emb_in)
    emb_out = _repack(emb_out)
    pos, negp = _sc_prod_kernel()(tgt, idx_neg, emb_out, bags)
    out = _tc_reduce(pos.reshape(_POS_ROWS, 128), negp.reshape(_NEG_ROWS, 128))
    return out.reshape(())


# speedup vs baseline: 1.8440x; 1.0013x over previous
import functools

import jax
import jax.numpy as jnp
from jax import lax
from jax.experimental import pallas as pl
from jax.experimental.pallas import tpu as pltpu
from jax.experimental.pallas import tpu_sc as plsc

B = 16384
D = 64
SEQ = 20
NNEG = 20
NL = 16

NC, NS = 2, 16
NW = NC * NS
BPW = B // NW
C = 64
NCHUNK = BPW // C
ROWS = C * SEQ


def _sc_bag_body(idx_in_hbm, emb_in_hbm, bags_hbm,
                 idx_in_v, rows_v, bag_v, sem):
    wid = lax.axis_index("s") * NC + lax.axis_index("c")

    def chunk_body(c, _):
        b0 = wid * BPW + c * C
        pltpu.sync_copy(idx_in_hbm.at[:, pl.ds(b0, C)], idx_in_v)
        cps = [pltpu.async_copy(emb_in_hbm.at[idx_in_v.at[l]],
                                rows_v.at[pl.ds(l * C, C)], sem)
               for l in range(SEQ)]
        for cp in cps:
            cp.wait()

        def bag_b(b, _):
            for k in range(D // NL):
                ds = pl.ds(k * NL, NL)
                bag = rows_v[b, ds]
                for l in range(1, SEQ):
                    bag = bag + rows_v[l * C + b, ds]
                bag_v[pl.ds(b * D + k * NL, NL)] = bag
            return 0

        lax.fori_loop(0, C, bag_b, 0)
        pltpu.sync_copy(bag_v, bags_hbm.at[pl.ds(b0 * D, C * D)])
        return 0

    lax.fori_loop(0, NCHUNK, chunk_body, 0)


def _sc_prod_body(tgt_hbm, idx_neg_hbm, emb_out_hbm, bags_hbm,
                  pos_hbm, negp_hbm,
                  idx_neg_v, idx_tgt_v, rows_v, out_rows_v, bag_v,
                  pos_v, negd_v, sem):
    wid = lax.axis_index("s") * NC + lax.axis_index("c")

    def chunk_body(c, _):
        b0 = wid * BPW + c * C
        pltpu.sync_copy(idx_neg_hbm.at[:, pl.ds(b0, C)], idx_neg_v)
        pltpu.sync_copy(tgt_hbm.at[pl.ds(b0, C)], idx_tgt_v)

        tgt_cp = pltpu.async_copy(emb_out_hbm.at[idx_tgt_v], out_rows_v, sem)
        ncps = [pltpu.async_copy(emb_out_hbm.at[idx_neg_v.at[n]],
                                 rows_v.at[pl.ds(n * C, C)], sem)
                for n in range(NNEG)]
        pltpu.sync_copy(bags_hbm.at[pl.ds(b0 * D, C * D)], bag_v)
        tgt_cp.wait()

        def pos_b(b, _):
            for k in range(D // NL):
                ds = pl.ds(k * NL, NL)
                prod = jnp.clip(bag_v[pl.ds(b * D + k * NL, NL)]
                                * out_rows_v[b, ds], -10.0, 10.0)
                pos_v[pl.ds(b * D + k * NL, NL)] = prod
            return 0

        lax.fori_loop(0, C, pos_b, 0)

        for cp in ncps:
            cp.wait()

        def neg_b(b, _):
            outs = [out_rows_v[b, pl.ds(k * NL, NL)] for k in range(D // NL)]
            for n in range(NNEG):
                acc = rows_v[n * C + b, pl.ds(0, NL)] * outs[0]
                for k in range(1, D // NL):
                    acc = acc + rows_v[n * C + b, pl.ds(k * NL, NL)] * outs[k]
                negd_v[pl.ds((b * NNEG + n) * NL, NL)] = acc
            return 0

        lax.fori_loop(0, C, neg_b, 0)

        pltpu.sync_copy(pos_v, pos_hbm.at[pl.ds(b0 * D, C * D)])
        pltpu.sync_copy(negd_v,
                        negp_hbm.at[pl.ds(b0 * NNEG * NL, C * NNEG * NL)])
        return 0

    lax.fori_loop(0, NCHUNK, chunk_body, 0)


_SC_MESH = dict(core_axis_name="c", subcore_axis_name="s",
                num_cores=NC, num_subcores=NS)


@functools.cache
def _sc_bag_kernel():
    return functools.partial(
        pl.kernel,
        out_type=[jax.ShapeDtypeStruct((B * D,), jnp.float32)],
        mesh=plsc.VectorSubcoreMesh(**_SC_MESH),
        scratch_types=[
            pltpu.VMEM((SEQ, C), jnp.int32),
            pltpu.VMEM((ROWS, D), jnp.float32),
            pltpu.VMEM((C * D,), jnp.float32),
            pltpu.SemaphoreType.DMA,
        ],
        compiler_params=pltpu.CompilerParams(use_tc_tiling_on_sc=False),
    )(_sc_bag_body)


@functools.cache
def _sc_prod_kernel():
    return functools.partial(
        pl.kernel,
        out_type=[jax.ShapeDtypeStruct((B * D,), jnp.float32),
                  jax.ShapeDtypeStruct((B * NNEG * NL,), jnp.float32)],
        mesh=plsc.VectorSubcoreMesh(**_SC_MESH),
        scratch_types=[
            pltpu.VMEM((NNEG, C), jnp.int32),
            pltpu.VMEM((C,), jnp.int32),
            pltpu.VMEM((ROWS, D), jnp.float32),
            pltpu.VMEM((C, D), jnp.float32),
            pltpu.VMEM((C * D,), jnp.float32),
            pltpu.VMEM((C * D,), jnp.float32),
            pltpu.VMEM((C * NNEG * NL,), jnp.float32),
            pltpu.SemaphoreType.DMA,
        ],
        compiler_params=pltpu.CompilerParams(use_tc_tiling_on_sc=False),
    )(_sc_prod_body)


_POS_ROWS = B * D // 128
_NEG_ROWS = B * NNEG * NL // 128
_TC_GRID = 8


def _tc_body(pos_ref, negp_ref, o_ref):
    i = pl.program_id(0)

    @pl.when(i == 0)
    def _():
        o_ref[0, 0] = 0.0

    def logsig_sum(x):
        return jnp.sum(jnp.minimum(x, 0.0) - jnp.log1p(jnp.exp(-jnp.abs(x))))

    r128 = lax.broadcasted_iota(jnp.int32, (128, 8), 0)
    r8 = lax.broadcasted_iota(jnp.int32, (128, 8), 1)
    sel = (r128 // NL == r8).astype(jnp.float32)
    g = lax.dot_general(negp_ref[...], sel, (((1,), (0,)), ((), ())),
                        precision=lax.Precision.HIGHEST)
    negd = jnp.clip(-g, -10.0, 10.0)
    o_ref[0, 0] += logsig_sum(pos_ref[...]) + logsig_sum(negd)

    @pl.when(i == pl.num_programs(0) - 1)
    def _():
        o_ref[0, 0] = -o_ref[0, 0] * (1.0 / B)


_tc_reduce = pl.pallas_call(
    _tc_body,
    grid=(_TC_GRID,),
    in_specs=[pl.BlockSpec((_POS_ROWS // _TC_GRID, 128), lambda i: (i, 0)),
              pl.BlockSpec((_NEG_ROWS // _TC_GRID, 128), lambda i: (i, 0))],
    out_specs=pl.BlockSpec((1, 1), lambda i: (0, 0),
                           memory_space=pltpu.SMEM),
    out_shape=jax.ShapeDtypeStruct((1, 1), jnp.float32),
)


VOCAB = 1000000
_RP_V = 32768
_RP_GRID = -(-VOCAB // _RP_V)
_RP_ROWS = _RP_GRID * _RP_V


def _repack_body(x_ref, o_ref):
    half = _RP_V // 2
    ya = jnp.transpose(x_ref[:, :half])
    yb = jnp.transpose(x_ref[:, half:])
    o_ref[...] = jnp.concatenate([ya, yb], axis=1)


_repack_tc = pl.pallas_call(
    _repack_body,
    grid=(_RP_GRID,),
    in_specs=[pl.BlockSpec((D, _RP_V), lambda i: (0, i))],
    out_specs=pl.BlockSpec((_RP_V // 2, 2 * D), lambda i: (i, 0)),
    out_shape=jax.ShapeDtypeStruct((_RP_ROWS // 2, 2 * D), jnp.float32),
)


def _repack(table):
    z = _repack_tc(table.T)
    return z.reshape(_RP_ROWS * D).reshape(_RP_ROWS, D)


def _remap(v):
    half = _RP_V // 2
    return (v // _RP_V) * _RP_V + (v % half) * 2 + (v // half) % 2


def kernel(inputs, target_word, negative_samples, emb_in, emb_out):
    idx_in = _remap(inputs.astype(jnp.int32).T)
    idx_neg = _remap(negative_samples.astype(jnp.int32).T)
    tgt = _remap(target_word.astype(jnp.int32))
    emb_in = _repack(emb_in)
    (bags,) = _sc_bag_kernel()(idx_in, emb_in)
    emb_out = _repack(emb_out)
    pos, negp = _sc_prod_kernel()(tgt, idx_neg, emb_out, bags)
    out = _tc_reduce(pos.reshape(_POS_ROWS, 128), negp.reshape(_NEG_ROWS, 128))
    return out.reshape(())
